# baseline (speedup 1.0000x reference)
import numpy as np
import jax
import jax.numpy as jnp
from jax.experimental import pallas as pl
from jax.experimental.pallas import tpu as pltpu

_SEQ = 336
_PRED = 96
_NBLK = 4
_NZOO = 3
_NB = 32
_NCH = 128
_KERN = 25
_OUT_LEN = _PRED * _NBLK
_PAD = (_KERN - 1) // 2
_AUG = _SEQ + 8
_BPS = 8

_HI = None


def _avg_matrix() -> np.ndarray:
    r = np.arange(_SEQ)[:, None]
    l = np.arange(_SEQ)[None, :]
    m = (np.abs(r - l) <= _PAD).astype(np.float32)
    m[:, 0] = np.maximum(0, _PAD + 1 - np.arange(_SEQ))
    m[:, _SEQ - 1] = np.maximum(0, np.arange(_SEQ) - (_SEQ - 2 - _PAD))
    return m / np.float32(_KERN)


def _mm(a, b, precision=_HI):
    return jax.lax.dot_general(
        a, b, (((1,), (0,)), ((), ())),
        preferred_element_type=jnp.float32, precision=precision)


def _fused_kernel(idx_ref, x_ref, ws_ref, bs_ref, m_ref, out_ref, g9_ref):
    b = pl.program_id(0)

    @pl.when(b == 0)
    def _compose():
        m = m_ref[...]
        lane0 = jax.lax.broadcasted_iota(jnp.int32, (_PRED, 8), 1) == 0
        ones2 = jnp.ones((2, 1), jnp.float32)
        diffs = [ws_ref[z, blk, 1] - ws_ref[z, blk, 0]
                 for z in range(_NZOO) for blk in range(_NBLK)]
        dm = _mm(jnp.concatenate(diffs, axis=0), m)
        ghat = []
        for z in range(_NZOO):
            glist = []
            for blk in range(_NBLK):
                row = _PRED * (z * _NBLK + blk)
                w = ws_ref[z, blk, 0] + dm[row:row + _PRED]
                beff = jax.lax.dot_general(
                    bs_ref[z, blk], ones2, (((0,), (0,)), ((), ())),
                    preferred_element_type=jnp.float32)
                bcol = jnp.where(lane0, jnp.broadcast_to(beff, (_PRED, 8)), 0.0)
                keep = _SEQ - _PRED * blk
                parts = ([jnp.zeros((_PRED, _PRED * blk), jnp.float32)]
                         if blk else [])
                parts += [w[:, :keep], bcol]
                g = jnp.concatenate(parts, axis=1)
                if blk:
                    g = g + _mm(w[:, keep:], jnp.concatenate(glist, axis=0))
                glist.append(g)
            gz = jnp.concatenate(glist, axis=0)
            g1 = jnp.sum(gz[:, :_SEQ], axis=1, keepdims=True)
            col337 = jax.lax.broadcasted_iota(
                jnp.int32, (_OUT_LEN, _AUG), 1) == _SEQ + 1
            gz = jnp.where(col337, jnp.broadcast_to(g1, (_OUT_LEN, _AUG)), gz)
            ghat.append(gz)
        for i in range(_NZOO):
            for j in range(_NZOO):
                g9_ref[i * _NZOO + j] = 0.5 * (ghat[i] + ghat[j])

    @pl.when(b > 0)
    def _apply():
        for s in range(_BPS):
            sample = (b - 1) * _BPS + s
            x = x_ref[s]
            pid = idx_ref[0, sample] * _NZOO + idx_ref[1, sample]
            g = g9_ref[pid]
            y = _mm(g[:, :_SEQ], x, precision=None)
            mean = jnp.mean(x, axis=0, keepdims=True)
            var = jnp.mean(x * x, axis=0, keepdims=True) - mean * mean
            std = jnp.sqrt(var + 1e-5)
            beta_col = jnp.broadcast_to(g[:, _SEQ:_SEQ + 1], (_OUT_LEN, _NCH))
            gone_col = jnp.broadcast_to(g[:, _SEQ + 1:_SEQ + 2],
                                        (_OUT_LEN, _NCH))
            mean_row = jnp.broadcast_to(mean, (_OUT_LEN, _NCH))
            std_row = jnp.broadcast_to(std, (_OUT_LEN, _NCH))
            out_ref[s] = y + (1.0 - gone_col) * mean_row + beta_col * std_row


def kernel(data, indices, x_mark_enc, x_dec, x_mark_dec, Ws, bs):
    del x_mark_enc, x_dec, x_mark_dec
    m = jnp.asarray(_avg_matrix())

    grid_spec = pltpu.PrefetchScalarGridSpec(
        num_scalar_prefetch=1,
        grid=(1 + _NB // _BPS,),
        in_specs=[
            pl.BlockSpec((_BPS, _SEQ, _NCH),
                         lambda b, idx: (jnp.maximum(b - 1, 0), 0, 0)),
            pl.BlockSpec((_NZOO, _NBLK, 2, _PRED, _SEQ),
                         lambda b, idx: (0, 0, 0, 0, 0)),
            pl.BlockSpec((_NZOO, _NBLK, 2, _PRED),
                         lambda b, idx: (0, 0, 0, 0)),
            pl.BlockSpec((_SEQ, _SEQ), lambda b, idx: (0, 0)),
        ],
        out_specs=pl.BlockSpec((_BPS, _OUT_LEN, _NCH),
                               lambda b, idx: (jnp.maximum(b - 1, 0), 0, 0)),
        scratch_shapes=[pltpu.VMEM((_NZOO * _NZOO, _OUT_LEN, _AUG),
                                   jnp.float32)],
    )
    out = pl.pallas_call(
        _fused_kernel,
        grid_spec=grid_spec,
        out_shape=jax.ShapeDtypeStruct((_NB, _OUT_LEN, _NCH), jnp.float32),
    )(indices.astype(jnp.int32), data, Ws, bs, m)
    return out


# speedup vs baseline: 1.0412x; 1.0412x over previous
import numpy as np
import jax
import jax.numpy as jnp
from jax.experimental import pallas as pl
from jax.experimental.pallas import tpu as pltpu

_SEQ = 336
_PRED = 96
_NBLK = 4
_NZOO = 3
_NB = 32
_NCH = 128
_KERN = 25
_OUT_LEN = _PRED * _NBLK
_PAD = (_KERN - 1) // 2
_AUG = _SEQ + 8
_BPS = 16

_HI = None


def _avg_matrix() -> np.ndarray:
    r = np.arange(_SEQ)[:, None]
    l = np.arange(_SEQ)[None, :]
    m = (np.abs(r - l) <= _PAD).astype(np.float32)
    m[:, 0] = np.maximum(0, _PAD + 1 - np.arange(_SEQ))
    m[:, _SEQ - 1] = np.maximum(0, np.arange(_SEQ) - (_SEQ - 2 - _PAD))
    return m / np.float32(_KERN)


def _mm(a, b, precision=_HI):
    return jax.lax.dot_general(
        a, b, (((1,), (0,)), ((), ())),
        preferred_element_type=jnp.float32, precision=precision)


def _fused_kernel(idx_ref, x_ref, ws_ref, bs_ref, m_ref, out_ref, g9_ref):
    b = pl.program_id(0)

    @pl.when(b == 0)
    def _compose():
        m = m_ref[...]
        lane0 = jax.lax.broadcasted_iota(jnp.int32, (_PRED, 8), 1) == 0
        ones2 = jnp.ones((2, 1), jnp.float32)
        diffs = [ws_ref[z, blk, 1] - ws_ref[z, blk, 0]
                 for z in range(_NZOO) for blk in range(_NBLK)]
        dm = _mm(jnp.concatenate(diffs, axis=0), m)
        ghat = []
        for z in range(_NZOO):
            glist = []
            for blk in range(_NBLK):
                row = _PRED * (z * _NBLK + blk)
                w = ws_ref[z, blk, 0] + dm[row:row + _PRED]
                beff = jax.lax.dot_general(
                    bs_ref[z, blk], ones2, (((0,), (0,)), ((), ())),
                    preferred_element_type=jnp.float32)
                bcol = jnp.where(lane0, jnp.broadcast_to(beff, (_PRED, 8)), 0.0)
                keep = _SEQ - _PRED * blk
                parts = ([jnp.zeros((_PRED, _PRED * blk), jnp.float32)]
                         if blk else [])
                parts += [w[:, :keep], bcol]
                g = jnp.concatenate(parts, axis=1)
                if blk:
                    g = g + _mm(w[:, keep:], jnp.concatenate(glist, axis=0))
                glist.append(g)
            gz = jnp.concatenate(glist, axis=0)
            g1 = jnp.sum(gz[:, :_SEQ], axis=1, keepdims=True)
            col337 = jax.lax.broadcasted_iota(
                jnp.int32, (_OUT_LEN, _AUG), 1) == _SEQ + 1
            gz = jnp.where(col337, jnp.broadcast_to(g1, (_OUT_LEN, _AUG)), gz)
            ghat.append(gz)
        for i in range(_NZOO):
            for j in range(_NZOO):
                g9_ref[i * _NZOO + j] = 0.5 * (ghat[i] + ghat[j])

    @pl.when(b > 0)
    def _apply():
        for s in range(_BPS):
            sample = (b - 1) * _BPS + s
            x = x_ref[s]
            pid = idx_ref[0, sample] * _NZOO + idx_ref[1, sample]
            g = g9_ref[pid]
            y = _mm(g[:, :_SEQ], x, precision=None)
            mean = jnp.mean(x, axis=0, keepdims=True)
            var = jnp.mean(x * x, axis=0, keepdims=True) - mean * mean
            std = jnp.sqrt(var + 1e-5)
            beta_col = jnp.broadcast_to(g[:, _SEQ:_SEQ + 1], (_OUT_LEN, _NCH))
            gone_col = jnp.broadcast_to(g[:, _SEQ + 1:_SEQ + 2],
                                        (_OUT_LEN, _NCH))
            mean_row = jnp.broadcast_to(mean, (_OUT_LEN, _NCH))
            std_row = jnp.broadcast_to(std, (_OUT_LEN, _NCH))
            out_ref[s] = y + (1.0 - gone_col) * mean_row + beta_col * std_row


def kernel(data, indices, x_mark_enc, x_dec, x_mark_dec, Ws, bs):
    del x_mark_enc, x_dec, x_mark_dec
    m = jnp.asarray(_avg_matrix())

    grid_spec = pltpu.PrefetchScalarGridSpec(
        num_scalar_prefetch=1,
        grid=(1 + _NB // _BPS,),
        in_specs=[
            pl.BlockSpec((_BPS, _SEQ, _NCH),
                         lambda b, idx: (jnp.maximum(b - 1, 0), 0, 0)),
            pl.BlockSpec((_NZOO, _NBLK, 2, _PRED, _SEQ),
                         lambda b, idx: (0, 0, 0, 0, 0)),
            pl.BlockSpec((_NZOO, _NBLK, 2, _PRED),
                         lambda b, idx: (0, 0, 0, 0)),
            pl.BlockSpec((_SEQ, _SEQ), lambda b, idx: (0, 0)),
        ],
        out_specs=pl.BlockSpec((_BPS, _OUT_LEN, _NCH),
                               lambda b, idx: (jnp.maximum(b - 1, 0), 0, 0)),
        scratch_shapes=[pltpu.VMEM((_NZOO * _NZOO, _OUT_LEN, _AUG),
                                   jnp.float32)],
    )
    out = pl.pallas_call(
        _fused_kernel,
        grid_spec=grid_spec,
        out_shape=jax.ShapeDtypeStruct((_NB, _OUT_LEN, _NCH), jnp.float32),
    )(indices.astype(jnp.int32), data, Ws, bs, m)
    return out
